# Initial kernel scaffold; baseline (speedup 1.0000x reference)
import jax
import jax.numpy as jnp
from jax.experimental import pallas as pl


def kernel(edge_index, x):
    raise NotImplementedError("write your pallas kernel here")


# speedup vs baseline: 4.9576x; 4.9576x over previous
import functools

import jax
import jax.numpy as jnp
from jax import lax
from jax.experimental import pallas as pl
from jax.experimental.pallas import tpu as pltpu
from jax.experimental.pallas import tpu_sc as plsc


def _sc_gather(idx, x, chunk):
    (B,) = idx.shape
    _, D = x.shape
    info = plsc.get_sparse_core_info()
    nc, ns = info.num_cores, info.num_subcores
    nw = nc * ns
    b_per_w = B // nw
    n_chunks = b_per_w // chunk
    assert b_per_w % chunk == 0 and B % nw == 0 and chunk % 8 == 0

    mesh = plsc.VectorSubcoreMesh(core_axis_name="c", subcore_axis_name="s")

    @functools.partial(
        pl.kernel,
        mesh=mesh,
        out_type=jax.ShapeDtypeStruct((B, D), x.dtype),
        scratch_types=[
            pltpu.VMEM((chunk,), jnp.int32),
            pltpu.VMEM((chunk, D), x.dtype),
            pltpu.SemaphoreType.DMA,
        ],
    )
    def k(idx_hbm, x_hbm, out_hbm, idx_v, rows_v, sem):
        wid = lax.axis_index("s") * nc + lax.axis_index("c")
        base = wid * b_per_w

        def body(i, carry):
            off = base + i * chunk
            pltpu.sync_copy(idx_hbm.at[pl.ds(off, chunk)], idx_v)
            pltpu.async_copy(x_hbm.at[idx_v], rows_v, sem).wait()
            pltpu.sync_copy(rows_v, out_hbm.at[pl.ds(off, chunk)])
            return carry

        lax.fori_loop(0, n_chunks, body, 0)

    return k(idx, x)


def kernel(edge_index, x):
    idx = edge_index[1]
    return _sc_gather(idx, x, chunk=400)


# speedup vs baseline: 5.7793x; 1.1657x over previous
import functools

import jax
import jax.numpy as jnp
from jax import lax
from jax.experimental import pallas as pl
from jax.experimental.pallas import tpu as pltpu
from jax.experimental.pallas import tpu_sc as plsc


def _sc_gather(idx, x, chunk):
    (B,) = idx.shape
    _, D = x.shape
    info = plsc.get_sparse_core_info()
    nc, ns = info.num_cores, info.num_subcores
    nw = nc * ns
    b_per_w = B // nw
    n_chunks = b_per_w // chunk
    assert b_per_w % chunk == 0 and B % nw == 0
    assert chunk % 8 == 0 and n_chunks % 2 == 0

    mesh = plsc.VectorSubcoreMesh(core_axis_name="c", subcore_axis_name="s")

    @functools.partial(
        pl.kernel,
        mesh=mesh,
        out_type=jax.ShapeDtypeStruct((B, D), x.dtype),
        scratch_types=[
            pltpu.VMEM((b_per_w,), jnp.int32),
            pltpu.VMEM((chunk, D), x.dtype),
            pltpu.VMEM((chunk, D), x.dtype),
            pltpu.SemaphoreType.DMA,
            pltpu.SemaphoreType.DMA,
            pltpu.SemaphoreType.DMA,
            pltpu.SemaphoreType.DMA,
        ],
    )
    def k(idx_hbm, x_hbm, out_hbm, idx_v, rows0, rows1, g0, g1, s0, s1):
        wid = lax.axis_index("s") * nc + lax.axis_index("c")
        base = wid * b_per_w
        rows = (rows0, rows1)
        gsem = (g0, g1)
        ssem = (s0, s1)

        def gather(j, b):
            return pltpu.make_async_copy(
                x_hbm.at[idx_v.at[pl.ds(j * chunk, chunk)]], rows[b], gsem[b]
            )

        def scatter(j, b):
            return pltpu.make_async_copy(
                rows[b], out_hbm.at[pl.ds(base + j * chunk, chunk)], ssem[b]
            )

        pltpu.sync_copy(idx_hbm.at[pl.ds(base, b_per_w)], idx_v)
        gather(0, 0).start()
        gather(1, 1).start()

        def body(i, carry):
            for b in range(2):
                j = 2 * i + b
                gather(j, b).wait()
                scatter(j, b).start()

                @pl.when(i < n_chunks // 2 - 1)
                def _():
                    scatter(j, b).wait()
                    gather(j + 2, b).start()

            return carry

        lax.fori_loop(0, n_chunks // 2, body, 0)
        scatter(n_chunks - 2, 0).wait()
        scatter(n_chunks - 1, 1).wait()

    return k(idx, x)


def kernel(edge_index, x):
    idx = edge_index[1]
    return _sc_gather(idx, x, chunk=200)


# speedup vs baseline: 8.2017x; 1.4192x over previous
import functools

import jax
import jax.numpy as jnp
from jax import lax
from jax.experimental import pallas as pl
from jax.experimental.pallas import tpu as pltpu
from jax.experimental.pallas import tpu_sc as plsc


def _sc_gather(idx, x, chunk):
    (B,) = idx.shape
    V, D = x.shape
    info = plsc.get_sparse_core_info()
    nc, ns = info.num_cores, info.num_subcores
    nw = nc * ns
    b_per_w = B // nw
    n_chunks = b_per_w // chunk
    assert b_per_w % chunk == 0 and B % nw == 0 and chunk % 8 == 0

    mesh = plsc.VectorSubcoreMesh(core_axis_name="c", subcore_axis_name="s")

    @functools.partial(
        pl.kernel,
        mesh=mesh,
        out_type=jax.ShapeDtypeStruct((B, D), x.dtype),
        scratch_types=[
            pltpu.VMEM_SHARED((V, D), x.dtype),
            pltpu.VMEM((b_per_w,), jnp.int32),
            pltpu.VMEM((chunk, D), x.dtype),
            pltpu.VMEM((chunk, D), x.dtype),
            pltpu.SemaphoreType.DMA,
            pltpu.SemaphoreType.DMA,
            pltpu.SemaphoreType.DMA,
            pltpu.SemaphoreType.DMA,
        ],
    )
    def k(idx_hbm, x_hbm, out_hbm, x_sp, idx_v, rows0, rows1, g0, g1, s0, s1):
        sid = lax.axis_index("s")
        wid = sid * nc + lax.axis_index("c")
        base = wid * b_per_w
        rows = (rows0, rows1)
        gsem = (g0, g1)
        ssem = (s0, s1)

        v_per_s = (V // ns) // 8 * 8
        pltpu.sync_copy(
            x_hbm.at[pl.ds(sid * v_per_s, v_per_s)],
            x_sp.at[pl.ds(sid * v_per_s, v_per_s)],
        )
        v_rem = V - ns * v_per_s
        if v_rem:

            @pl.when(sid == 0)
            def _():
                pltpu.sync_copy(
                    x_hbm.at[pl.ds(ns * v_per_s, v_rem)],
                    x_sp.at[pl.ds(ns * v_per_s, v_rem)],
                )

        def gather(j, b):
            return pltpu.make_async_copy(
                x_sp.at[idx_v.at[pl.ds(j * chunk, chunk)]], rows[b], gsem[b]
            )

        def scatter(j, b):
            return pltpu.make_async_copy(
                rows[b], out_hbm.at[pl.ds(base + j * chunk, chunk)], ssem[b]
            )

        pltpu.sync_copy(idx_hbm.at[pl.ds(base, b_per_w)], idx_v)
        plsc.subcore_barrier()
        gather(0, 0).start()
        gather(1, 1).start()

        def body(i, carry):
            for b in range(2):
                j = 2 * i + b
                gather(j, b).wait()
                scatter(j, b).start()

                @pl.when(j + 2 < n_chunks)
                def _():
                    scatter(j, b).wait()
                    gather(j + 2, b).start()

            return carry

        lax.fori_loop(0, n_chunks // 2, body, 0)
        if n_chunks % 2:
            jl = n_chunks - 1
            gather(jl, 0).wait()
            scatter(jl, 0).start()
            scatter(jl - 1, 1).wait()
            scatter(jl, 0).wait()
        else:
            scatter(n_chunks - 2, 0).wait()
            scatter(n_chunks - 1, 1).wait()

    return k(idx, x)


def kernel(edge_index, x):
    idx = edge_index[1]
    return _sc_gather(idx, x, chunk=80)


# speedup vs baseline: 8.2227x; 1.0026x over previous
import functools

import jax
import jax.numpy as jnp
from jax import lax
from jax.experimental import pallas as pl
from jax.experimental.pallas import tpu as pltpu
from jax.experimental.pallas import tpu_sc as plsc


def _sc_gather(idx, x, chunk, hbm_chunks):
    (B,) = idx.shape
    V, D = x.shape
    info = plsc.get_sparse_core_info()
    nc, ns = info.num_cores, info.num_subcores
    nw = nc * ns
    b_per_w = B // nw
    n_chunks = b_per_w // chunk
    assert b_per_w % chunk == 0 and B % nw == 0 and chunk % 8 == 0
    assert hbm_chunks % 2 == 0 and hbm_chunks + 2 <= n_chunks

    mesh = plsc.VectorSubcoreMesh(core_axis_name="c", subcore_axis_name="s")

    @functools.partial(
        pl.kernel,
        mesh=mesh,
        out_type=jax.ShapeDtypeStruct((B, D), x.dtype),
        scratch_types=[
            pltpu.VMEM_SHARED((V, D), x.dtype),
            pltpu.VMEM((b_per_w,), jnp.int32),
            pltpu.VMEM((chunk, D), x.dtype),
            pltpu.VMEM((chunk, D), x.dtype),
            pltpu.SemaphoreType.DMA,
            pltpu.SemaphoreType.DMA,
            pltpu.SemaphoreType.DMA,
            pltpu.SemaphoreType.DMA,
            pltpu.SemaphoreType.DMA,
        ],
    )
    def k(idx_hbm, x_hbm, out_hbm, x_sp, idx_v, rows0, rows1, g0, g1, s0, s1, st):
        sid = lax.axis_index("s")
        wid = sid * nc + lax.axis_index("c")
        base = wid * b_per_w
        rows = (rows0, rows1)
        gsem = (g0, g1)
        ssem = (s0, s1)

        v_per_s = (V // ns) // 8 * 8
        stage = pltpu.make_async_copy(
            x_hbm.at[pl.ds(sid * v_per_s, v_per_s)],
            x_sp.at[pl.ds(sid * v_per_s, v_per_s)],
            st,
        )
        stage.start()
        v_rem = V - ns * v_per_s
        if v_rem:
            stage_rem = pltpu.make_async_copy(
                x_hbm.at[pl.ds(ns * v_per_s, v_rem)],
                x_sp.at[pl.ds(ns * v_per_s, v_rem)],
                st,
            )

            @pl.when(sid == 0)
            def _():
                stage_rem.start()

        pltpu.sync_copy(idx_hbm.at[pl.ds(base, b_per_w)], idx_v)

        def gather_hbm(j, b):
            return pltpu.make_async_copy(
                x_hbm.at[idx_v.at[pl.ds(j * chunk, chunk)]], rows[b], gsem[b]
            )

        def gather(j, b):
            return pltpu.make_async_copy(
                x_sp.at[idx_v.at[pl.ds(j * chunk, chunk)]], rows[b], gsem[b]
            )

        def scatter(j, b):
            return pltpu.make_async_copy(
                rows[b], out_hbm.at[pl.ds(base + j * chunk, chunk)], ssem[b]
            )

        gather_hbm(0, 0).start()
        gather_hbm(1, 1).start()

        def body1(i, carry):
            for b in range(2):
                j = 2 * i + b
                gather_hbm(j, b).wait()
                scatter(j, b).start()

                @pl.when(j + 2 < hbm_chunks)
                def _():
                    scatter(j, b).wait()
                    gather_hbm(j + 2, b).start()

            return carry

        lax.fori_loop(0, hbm_chunks // 2, body1, 0)

        stage.wait()
        if v_rem:

            @pl.when(sid == 0)
            def _():
                stage_rem.wait()

        plsc.subcore_barrier()

        scatter(hbm_chunks - 2, 0).wait()
        gather(hbm_chunks, 0).start()
        scatter(hbm_chunks - 1, 1).wait()
        gather(hbm_chunks + 1, 1).start()

        def body2(i2, carry):
            for b in range(2):
                j = hbm_chunks + 2 * i2 + b
                gather(j, b).wait()
                scatter(j, b).start()

                @pl.when(j + 2 < n_chunks)
                def _():
                    scatter(j, b).wait()
                    gather(j + 2, b).start()

            return carry

        lax.fori_loop(0, (n_chunks - hbm_chunks) // 2, body2, 0)
        if (n_chunks - hbm_chunks) % 2:
            jl = n_chunks - 1
            gather(jl, 0).wait()
            scatter(jl, 0).start()
            scatter(jl - 1, 1).wait()
            scatter(jl, 0).wait()
        else:
            scatter(n_chunks - 2, 0).wait()
            scatter(n_chunks - 1, 1).wait()

    return k(idx, x)


def kernel(edge_index, x):
    idx = edge_index[1]
    return _sc_gather(idx, x, chunk=80, hbm_chunks=8)
